# Initial kernel scaffold; baseline (speedup 1.0000x reference)
import jax
import jax.numpy as jnp
from jax.experimental import pallas as pl


def kernel(x, edge_index, p_w1, p_b1, pln_g, pln_b, p_w2, p_b2, c1_w, c1_b, ln1_g, ln1_b, c2_w, c2_b, ln2_g, ln2_b, c3_w, c3_b, ln3_g, ln3_b, res_w, res_b):
    raise NotImplementedError("write your pallas kernel here")


# speedup vs baseline: 5.5997x; 5.5997x over previous
import functools

import jax
import jax.numpy as jnp
from jax import lax
from jax.experimental import pallas as pl
from jax.experimental.pallas import tpu as pltpu
from jax.experimental.pallas import tpu_sc as plsc

NC = 2
NS = 16
EC = 128
FC = 128


def _mesh():
    return plsc.VectorSubcoreMesh(
        core_axis_name="c", subcore_axis_name="s", num_cores=NC, num_subcores=NS
    )


def _ceil_to(a, m):
    return (a + m - 1) // m * m


def _sc_degree(dsts, z1d, NP, EP):
    EPC = EP // NC
    EPT = EPC // NS
    NCH = EPT // EC
    RPT = NP // NS

    @functools.partial(
        pl.kernel,
        out_type=(
            jax.ShapeDtypeStruct((NP,), jnp.float32),
            jax.ShapeDtypeStruct((NP,), jnp.float32),
        ),
        mesh=_mesh(),
        scratch_types=[
            pltpu.VMEM((EC,), jnp.int32),
            pltpu.VMEM((EC,), jnp.float32),
            pltpu.VMEM_SHARED((NP,), jnp.float32),
        ],
    )
    def k(dst_hbm, z_hbm, out0, out1, didx, ones_v, sdeg):
        cid = lax.axis_index("c")
        sid = lax.axis_index("s")
        for j in range(EC // 16):
            ones_v[pl.ds(j * 16, 16)] = jnp.full((16,), 1.0, jnp.float32)

        @pl.when(sid == 0)
        def _():
            pltpu.sync_copy(z_hbm, sdeg)

        plsc.subcore_barrier()

        def body(i, carry):
            base = cid * EPC + sid * EPT + i * EC
            pltpu.sync_copy(dst_hbm.at[pl.ds(base, EC)], didx)
            pltpu.sync_copy(ones_v, sdeg.at[didx], add=True)
            return carry

        lax.fori_loop(0, NCH, body, 0)
        plsc.subcore_barrier()

        @pl.when(cid == 0)
        def _():
            pltpu.sync_copy(
                sdeg.at[pl.ds(sid * RPT, RPT)], out0.at[pl.ds(sid * RPT, RPT)]
            )

        @pl.when(cid == 1)
        def _():
            pltpu.sync_copy(
                sdeg.at[pl.ds(sid * RPT, RPT)], out1.at[pl.ds(sid * RPT, RPT)]
            )

    return k(dsts, z1d)


def _sc_agg512(y_chunks, srcs, dsts, zrows, NP, EP):
    EPT = EP // NS
    NCH = EPT // EC
    RPT = NP // NS

    @functools.partial(
        pl.kernel,
        out_type=tuple(
            jax.ShapeDtypeStruct((NP, FC), jnp.float32) for _ in range(4)
        ),
        mesh=_mesh(),
        scratch_types=[
            pltpu.VMEM((EC,), jnp.int32),
            pltpu.VMEM((EC,), jnp.int32),
            pltpu.VMEM((EC, FC), jnp.float32),
            pltpu.VMEM_SHARED((NP, FC), jnp.float32),
            pltpu.SemaphoreType.DMA,
        ],
    )
    def k(y0, y1, y2, y3, src_hbm, dst_hbm, z_hbm, o0, o1, o2, o3,
          sidx, didx, rows, acc, sem):
        cid = lax.axis_index("c")
        sid = lax.axis_index("s")

        def one_chunk(y_hbm, out_hbm):
            @pl.when(sid == 0)
            def _():
                pltpu.sync_copy(z_hbm, acc)

            plsc.subcore_barrier()

            def body(i, carry):
                base = sid * EPT + i * EC
                pltpu.sync_copy(src_hbm.at[pl.ds(base, EC)], sidx)
                pltpu.sync_copy(dst_hbm.at[pl.ds(base, EC)], didx)
                pltpu.async_copy(y_hbm.at[sidx], rows, sem).wait()
                pltpu.sync_copy(rows, acc.at[didx], add=True)
                return carry

            lax.fori_loop(0, NCH, body, 0)
            plsc.subcore_barrier()
            pltpu.sync_copy(
                acc.at[pl.ds(sid * RPT, RPT)],
                out_hbm.at[pl.ds(sid * RPT, RPT)],
            )
            plsc.subcore_barrier()

        @pl.when(cid == 0)
        def _():
            one_chunk(y0, o0)
            one_chunk(y1, o1)

        @pl.when(cid == 1)
        def _():
            one_chunk(y2, o2)
            one_chunk(y3, o3)

    return k(*y_chunks, srcs, dsts, zrows)


def _sc_agg128(y, srcs, dsts, zrows, NP, EP):
    EPC = EP // NC
    EPT = EPC // NS
    NCH = EPT // EC
    RPT = NP // NS

    @functools.partial(
        pl.kernel,
        out_type=(
            jax.ShapeDtypeStruct((NP, FC), jnp.float32),
            jax.ShapeDtypeStruct((NP, FC), jnp.float32),
        ),
        mesh=_mesh(),
        scratch_types=[
            pltpu.VMEM((EC,), jnp.int32),
            pltpu.VMEM((EC,), jnp.int32),
            pltpu.VMEM((EC, FC), jnp.float32),
            pltpu.VMEM_SHARED((NP, FC), jnp.float32),
            pltpu.SemaphoreType.DMA,
        ],
    )
    def k(y_hbm, src_hbm, dst_hbm, z_hbm, o0, o1, sidx, didx, rows, acc, sem):
        cid = lax.axis_index("c")
        sid = lax.axis_index("s")

        @pl.when(sid == 0)
        def _():
            pltpu.sync_copy(z_hbm, acc)

        plsc.subcore_barrier()

        def body(i, carry):
            base = cid * EPC + sid * EPT + i * EC
            pltpu.sync_copy(src_hbm.at[pl.ds(base, EC)], sidx)
            pltpu.sync_copy(dst_hbm.at[pl.ds(base, EC)], didx)
            pltpu.async_copy(y_hbm.at[sidx], rows, sem).wait()
            pltpu.sync_copy(rows, acc.at[didx], add=True)
            return carry

        lax.fori_loop(0, NCH, body, 0)
        plsc.subcore_barrier()

        @pl.when(cid == 0)
        def _():
            pltpu.sync_copy(
                acc.at[pl.ds(sid * RPT, RPT)], o0.at[pl.ds(sid * RPT, RPT)]
            )

        @pl.when(cid == 1)
        def _():
            pltpu.sync_copy(
                acc.at[pl.ds(sid * RPT, RPT)], o1.at[pl.ds(sid * RPT, RPT)]
            )

    return k(y, srcs, dsts, zrows)


def _ln_rows(h, g, b, eps=1e-5):
    m = jnp.mean(h, axis=-1, keepdims=True)
    v = jnp.mean((h - m) * (h - m), axis=-1, keepdims=True)
    return (h - m) * lax.rsqrt(v + eps) * g + b


def _dinv_block(d0, d1):
    return lax.rsqrt(d0[...] + d1[...])


def _tc_proj(x, w1, b1, g1, bb1, w2, b2, cw, d0, d1, NP, BN):
    HC = w1.shape[1]
    grid = NP // BN

    def body(x_ref, w1_ref, b1_ref, g1_ref, bb1_ref, w2_ref, b2_ref, cw_ref,
             d0_ref, d1_ref, idn_ref, y_ref):
        h = jnp.dot(x_ref[...], w1_ref[...], preferred_element_type=jnp.float32)
        h = jnp.maximum(h + b1_ref[...], 0.0)
        h = _ln_rows(h, g1_ref[...], bb1_ref[...])
        h = jnp.dot(h, w2_ref[...], preferred_element_type=jnp.float32)
        h = h + b2_ref[...]
        idn_ref[...] = h
        dinv = _dinv_block(d0_ref, d1_ref)
        y = jnp.dot(h, cw_ref[...], preferred_element_type=jnp.float32)
        y = y * dinv
        for kk in range(4):
            y_ref[kk] = y[:, kk * FC:(kk + 1) * FC]

    full = lambda r, c: pl.BlockSpec((r, c), lambda n: (0, 0))
    return pl.pallas_call(
        body,
        grid=(grid,),
        in_specs=[
            pl.BlockSpec((BN, x.shape[1]), lambda n: (n, 0)),
            full(*w1.shape),
            pl.BlockSpec((1, HC), lambda n: (0, 0)),
            pl.BlockSpec((1, HC), lambda n: (0, 0)),
            pl.BlockSpec((1, HC), lambda n: (0, 0)),
            full(*w2.shape),
            pl.BlockSpec((1, HC), lambda n: (0, 0)),
            full(*cw.shape),
            pl.BlockSpec((BN, 1), lambda n: (n, 0)),
            pl.BlockSpec((BN, 1), lambda n: (n, 0)),
        ],
        out_specs=[
            pl.BlockSpec((BN, HC), lambda n: (n, 0)),
            pl.BlockSpec((4, BN, FC), lambda n: (0, n, 0)),
        ],
        out_shape=[
            jax.ShapeDtypeStruct((NP, HC), jnp.float32),
            jax.ShapeDtypeStruct((4, NP, FC), jnp.float32),
        ],
    )(x, w1, b1, g1, bb1, w2, b2, cw,
      d0, d1)


def _tc_mid(aggs, cb, g, b, idn, w, d0, d1, NP, BN):
    HC = idn.shape[1]
    OC2 = w.shape[1]
    grid = NP // BN
    nyc = OC2 // FC

    def body(a_ref, cb_ref, g_ref, b_ref, idn_ref, w_ref, d0_ref, d1_ref,
             h_ref, y_ref):
        agg = jnp.concatenate([a_ref[kk] for kk in range(4)], axis=-1)
        dinv = _dinv_block(d0_ref, d1_ref)
        h = agg * dinv + cb_ref[...]
        h = jnp.maximum(h, 0.0)
        h = _ln_rows(h, g_ref[...], b_ref[...])
        h = h + idn_ref[...]
        h_ref[...] = h
        y = jnp.dot(h, w_ref[...], preferred_element_type=jnp.float32)
        y = y * dinv
        if nyc > 1:
            for kk in range(nyc):
                y_ref[kk] = y[:, kk * FC:(kk + 1) * FC]
        else:
            y_ref[...] = y

    y_spec = (pl.BlockSpec((nyc, BN, FC), lambda n: (0, n, 0)) if nyc > 1
              else pl.BlockSpec((BN, FC), lambda n: (n, 0)))
    y_shape = ((nyc, NP, FC) if nyc > 1 else (NP, FC))
    return pl.pallas_call(
        body,
        grid=(grid,),
        in_specs=[
            pl.BlockSpec((4, BN, FC), lambda n: (0, n, 0)),
            pl.BlockSpec((1, HC), lambda n: (0, 0)),
            pl.BlockSpec((1, HC), lambda n: (0, 0)),
            pl.BlockSpec((1, HC), lambda n: (0, 0)),
            pl.BlockSpec((BN, HC), lambda n: (n, 0)),
            pl.BlockSpec(w.shape, lambda n: (0, 0)),
            pl.BlockSpec((BN, 1), lambda n: (n, 0)),
            pl.BlockSpec((BN, 1), lambda n: (n, 0)),
        ],
        out_specs=[
            pl.BlockSpec((BN, HC), lambda n: (n, 0)),
            y_spec,
        ],
        out_shape=[
            jax.ShapeDtypeStruct((NP, HC), jnp.float32),
            jax.ShapeDtypeStruct(y_shape, jnp.float32),
        ],
    )(aggs, cb, g, b, idn, w,
      d0, d1)


def _tc_final(p0, p1, cb, g, b, idn, rw, rb, d0, d1, NP, BN):
    HC = idn.shape[1]
    OC2 = rw.shape[1]
    grid = NP // BN

    def body(p0_ref, p1_ref, cb_ref, g_ref, b_ref, idn_ref, rw_ref, rb_ref,
             d0_ref, d1_ref, o_ref):
        dinv = _dinv_block(d0_ref, d1_ref)
        h = (p0_ref[...] + p1_ref[...]) * dinv + cb_ref[...]
        h = jnp.maximum(h, 0.0)
        h = _ln_rows(h, g_ref[...], b_ref[...])
        res = jnp.dot(idn_ref[...], rw_ref[...],
                      preferred_element_type=jnp.float32)
        o_ref[...] = h + res + rb_ref[...]

    return pl.pallas_call(
        body,
        grid=(grid,),
        in_specs=[
            pl.BlockSpec((BN, OC2), lambda n: (n, 0)),
            pl.BlockSpec((BN, OC2), lambda n: (n, 0)),
            pl.BlockSpec((1, OC2), lambda n: (0, 0)),
            pl.BlockSpec((1, OC2), lambda n: (0, 0)),
            pl.BlockSpec((1, OC2), lambda n: (0, 0)),
            pl.BlockSpec((BN, HC), lambda n: (n, 0)),
            pl.BlockSpec(rw.shape, lambda n: (0, 0)),
            pl.BlockSpec((1, OC2), lambda n: (0, 0)),
            pl.BlockSpec((BN, 1), lambda n: (n, 0)),
            pl.BlockSpec((BN, 1), lambda n: (n, 0)),
        ],
        out_specs=pl.BlockSpec((BN, OC2), lambda n: (n, 0)),
        out_shape=jax.ShapeDtypeStruct((NP, OC2), jnp.float32),
    )(p0, p1, cb, g, b, idn, rw, rb,
      d0, d1)


def kernel(x, edge_index, p_w1, p_b1, pln_g, pln_b, p_w2, p_b2,
           c1_w, c1_b, ln1_g, ln1_b, c2_w, c2_b, ln2_g, ln2_b,
           c3_w, c3_b, ln3_g, ln3_b, res_w, res_b):
    N, IC = x.shape
    HC = p_w1.shape[1]
    OC = c3_w.shape[1]
    E = edge_index.shape[1]
    BN = 512
    NP = _ceil_to(N, BN)
    E2 = E + NP
    EP = _ceil_to(E2, NC * NS * EC)
    pad = EP - E2

    i32 = jnp.int32
    loop = jnp.arange(NP, dtype=i32)
    srcs = jnp.concatenate(
        [edge_index[0], loop, jnp.zeros((pad,), i32)])
    dsts = jnp.concatenate(
        [edge_index[1], loop, jnp.full((pad,), NP - 1, i32)])
    x_pad = jnp.pad(x, ((0, NP - N), (0, 0)))

    zrows = jnp.zeros((NP, FC), jnp.float32)
    z1d = jnp.zeros((NP,), jnp.float32)

    b2 = lambda v: v.reshape(1, -1)

    d0f, d1f = _sc_degree(dsts, z1d, NP, EP)
    d0 = d0f.reshape(NP, 1)
    d1 = d1f.reshape(NP, 1)

    idn, y1 = _tc_proj(x_pad, p_w1, b2(p_b1), b2(pln_g), b2(pln_b),
                       p_w2, b2(p_b2), c1_w, d0, d1, NP, BN)

    a = _sc_agg512([y1[0], y1[1], y1[2], y1[3]], srcs, dsts, zrows, NP, EP)
    agg1 = jnp.stack(a)

    idn2, y2 = _tc_mid(agg1, b2(c1_b), b2(ln1_g), b2(ln1_b), idn, c2_w,
                       d0, d1, NP, BN)

    a = _sc_agg512([y2[0], y2[1], y2[2], y2[3]], srcs, dsts, zrows, NP, EP)
    agg2 = jnp.stack(a)

    idn3, y3 = _tc_mid(agg2, b2(c2_b), b2(ln2_g), b2(ln2_b), idn2, c3_w,
                       d0, d1, NP, BN)

    p0, p1 = _sc_agg128(y3, srcs, dsts, zrows, NP, EP)

    out = _tc_final(p0, p1, b2(c3_b), b2(ln3_g), b2(ln3_b), idn3,
                    res_w, b2(res_b), d0, d1, NP, BN)
    return out[:N]


# speedup vs baseline: 13.4970x; 2.4103x over previous
import functools

import jax
import jax.numpy as jnp
from jax import lax
from jax.experimental import pallas as pl
from jax.experimental.pallas import tpu as pltpu
from jax.experimental.pallas import tpu_sc as plsc

NC = 2
NS = 16
EC = 128
FC = 128


def _mesh():
    return plsc.VectorSubcoreMesh(
        core_axis_name="c", subcore_axis_name="s", num_cores=NC, num_subcores=NS
    )


def _ceil_to(a, m):
    return (a + m - 1) // m * m


def _sc_degree(dsts, z1d, NP, EP):
    EPC = EP // NC
    EPT = EPC // NS
    NCH = EPT // EC
    RPT = NP // NS

    @functools.partial(
        pl.kernel,
        out_type=(
            jax.ShapeDtypeStruct((NP,), jnp.float32),
            jax.ShapeDtypeStruct((NP,), jnp.float32),
        ),
        mesh=_mesh(),
        scratch_types=[
            pltpu.VMEM((EC,), jnp.int32),
            pltpu.VMEM((EC,), jnp.float32),
            pltpu.VMEM_SHARED((NP,), jnp.float32),
        ],
    )
    def k(dst_hbm, z_hbm, out0, out1, didx, ones_v, sdeg):
        cid = lax.axis_index("c")
        sid = lax.axis_index("s")
        for j in range(EC // 16):
            ones_v[pl.ds(j * 16, 16)] = jnp.full((16,), 1.0, jnp.float32)

        @pl.when(sid == 0)
        def _():
            pltpu.sync_copy(z_hbm, sdeg)

        plsc.subcore_barrier()

        def body(i, carry):
            base = cid * EPC + sid * EPT + i * EC
            pltpu.sync_copy(dst_hbm.at[pl.ds(base, EC)], didx)
            pltpu.sync_copy(ones_v, sdeg.at[didx], add=True)
            return carry

        lax.fori_loop(0, NCH, body, 0)
        plsc.subcore_barrier()

        @pl.when(cid == 0)
        def _():
            pltpu.sync_copy(
                sdeg.at[pl.ds(sid * RPT, RPT)], out0.at[pl.ds(sid * RPT, RPT)]
            )

        @pl.when(cid == 1)
        def _():
            pltpu.sync_copy(
                sdeg.at[pl.ds(sid * RPT, RPT)], out1.at[pl.ds(sid * RPT, RPT)]
            )

    return k(dsts, z1d)


def _edge_pipeline(y_hbm, src_hbm, dst_hbm, acc, sidx, didx, rows,
                   isem, gsem, ssem, base0, NCH):
    NB8, TAIL = NCH // 8, NCH % 8

    def idxload(i, s):
        base = base0 + i * EC
        pltpu.async_copy(src_hbm.at[pl.ds(base, EC)], sidx.at[s], isem[s])
        pltpu.async_copy(dst_hbm.at[pl.ds(base, EC)], didx.at[s], isem[s])

    def idxwait(s):
        pltpu.make_async_copy(
            src_hbm.at[pl.ds(0, EC)], sidx.at[s], isem[s]).wait()
        pltpu.make_async_copy(
            dst_hbm.at[pl.ds(0, EC)], didx.at[s], isem[s]).wait()

    def gather(s, p):
        pltpu.async_copy(y_hbm.at[sidx.at[s]], rows.at[p], gsem[p])

    def gatherwait(p):
        pltpu.make_async_copy(
            y_hbm.at[sidx.at[0]], rows.at[p], gsem[p]).wait()

    def scatter(s, p):
        pltpu.async_copy(rows.at[p], acc.at[didx.at[s]], ssem[p], add=True)

    def scatterwait(p):
        pltpu.make_async_copy(
            rows.at[p], acc.at[didx.at[0]], ssem[p]).wait()

    for s in range(6):
        idxload(jnp.int32(s), s)
    idxwait(0)
    gather(0, 0)

    def substep(i, b, static):
        s6, s1 = (b + 6) % 8, (b + 1) % 8
        p, p1 = b % 2, (b + 1) % 2

        def guard(cond, fn):
            if static:
                if cond:
                    fn()
            else:
                pl.when(cond)(fn)

        guard(i + 6 < NCH, lambda: idxload(i + 6, s6))

        def prep_next():
            idxwait(s1)
            guard(i >= 1, lambda: scatterwait(p1))
            gather(s1, p1)

        guard(i + 1 < NCH, prep_next)
        gatherwait(p)
        scatter(b, p)

    def body(j, carry):
        for b in range(8):
            substep(j * 8 + b, b, False)
        return carry

    lax.fori_loop(0, NB8, body, 0)
    for b in range(TAIL):
        substep(NB8 * 8 + b, b, True)
    scatterwait(0)
    scatterwait(1)


def _sc_agg512(y_chunks, srcs, dsts, zrows, NP, EP):
    EPT = EP // NS
    NCH = EPT // EC
    RPT = NP // NS

    @functools.partial(
        pl.kernel,
        out_type=tuple(
            jax.ShapeDtypeStruct((NP, FC), jnp.float32) for _ in range(4)
        ),
        mesh=_mesh(),
        scratch_types=[
            pltpu.VMEM((8, EC), jnp.int32),
            pltpu.VMEM((8, EC), jnp.int32),
            pltpu.VMEM((2, EC, FC), jnp.float32),
            pltpu.VMEM_SHARED((NP, FC), jnp.float32),
            [pltpu.SemaphoreType.DMA] * 8,
            [pltpu.SemaphoreType.DMA] * 2,
            [pltpu.SemaphoreType.DMA] * 2,
        ],
    )
    def k(y0, y1, y2, y3, src_hbm, dst_hbm, z_hbm, o0, o1, o2, o3,
          sidx, didx, rows, acc, isem, gsem, ssem):
        cid = lax.axis_index("c")
        sid = lax.axis_index("s")

        def one_chunk(y_hbm, out_hbm):
            pltpu.sync_copy(
                z_hbm.at[pl.ds(sid * RPT, RPT)], acc.at[pl.ds(sid * RPT, RPT)]
            )
            plsc.subcore_barrier()
            _edge_pipeline(y_hbm, src_hbm, dst_hbm, acc, sidx, didx, rows,
                           isem, gsem, ssem, sid * EPT, NCH)
            plsc.subcore_barrier()
            pltpu.sync_copy(
                acc.at[pl.ds(sid * RPT, RPT)],
                out_hbm.at[pl.ds(sid * RPT, RPT)],
            )
            plsc.subcore_barrier()

        @pl.when(cid == 0)
        def _():
            one_chunk(y0, o0)
            one_chunk(y1, o1)

        @pl.when(cid == 1)
        def _():
            one_chunk(y2, o2)
            one_chunk(y3, o3)

    return k(*y_chunks, srcs, dsts, zrows)


def _sc_agg128(y, srcs, dsts, zrows, NP, EP):
    EPC = EP // NC
    EPT = EPC // NS
    NCH = EPT // EC
    RPT = NP // NS

    @functools.partial(
        pl.kernel,
        out_type=(
            jax.ShapeDtypeStruct((NP, FC), jnp.float32),
            jax.ShapeDtypeStruct((NP, FC), jnp.float32),
        ),
        mesh=_mesh(),
        scratch_types=[
            pltpu.VMEM((8, EC), jnp.int32),
            pltpu.VMEM((8, EC), jnp.int32),
            pltpu.VMEM((2, EC, FC), jnp.float32),
            pltpu.VMEM_SHARED((NP, FC), jnp.float32),
            [pltpu.SemaphoreType.DMA] * 8,
            [pltpu.SemaphoreType.DMA] * 2,
            [pltpu.SemaphoreType.DMA] * 2,
        ],
    )
    def k(y_hbm, src_hbm, dst_hbm, z_hbm, o0, o1,
          sidx, didx, rows, acc, isem, gsem, ssem):
        cid = lax.axis_index("c")
        sid = lax.axis_index("s")
        pltpu.sync_copy(
            z_hbm.at[pl.ds(sid * RPT, RPT)], acc.at[pl.ds(sid * RPT, RPT)]
        )
        plsc.subcore_barrier()
        _edge_pipeline(y_hbm, src_hbm, dst_hbm, acc, sidx, didx, rows,
                       isem, gsem, ssem, cid * EPC + sid * EPT, NCH)
        plsc.subcore_barrier()

        @pl.when(cid == 0)
        def _():
            pltpu.sync_copy(
                acc.at[pl.ds(sid * RPT, RPT)], o0.at[pl.ds(sid * RPT, RPT)]
            )

        @pl.when(cid == 1)
        def _():
            pltpu.sync_copy(
                acc.at[pl.ds(sid * RPT, RPT)], o1.at[pl.ds(sid * RPT, RPT)]
            )

    return k(y, srcs, dsts, zrows)


def _ln_rows(h, g, b, eps=1e-5):
    m = jnp.mean(h, axis=-1, keepdims=True)
    v = jnp.mean((h - m) * (h - m), axis=-1, keepdims=True)
    return (h - m) * lax.rsqrt(v + eps) * g + b


def _dinv_block(d0, d1):
    return lax.rsqrt(d0[...] + d1[...])


def _tc_proj(x, w1, b1, g1, bb1, w2, b2, cw, d0, d1, NP, BN):
    HC = w1.shape[1]
    grid = NP // BN

    def body(x_ref, w1_ref, b1_ref, g1_ref, bb1_ref, w2_ref, b2_ref, cw_ref,
             d0_ref, d1_ref, idn_ref, *y_refs):
        h = jnp.dot(x_ref[...], w1_ref[...], preferred_element_type=jnp.float32)
        h = jnp.maximum(h + b1_ref[...], 0.0)
        h = _ln_rows(h, g1_ref[...], bb1_ref[...])
        h = jnp.dot(h, w2_ref[...], preferred_element_type=jnp.float32)
        h = h + b2_ref[...]
        idn_ref[...] = h
        dinv = _dinv_block(d0_ref, d1_ref)
        y = jnp.dot(h, cw_ref[...], preferred_element_type=jnp.float32)
        y = y * dinv
        for kk in range(4):
            y_refs[kk][...] = y[:, kk * FC:(kk + 1) * FC]

    full = lambda r, c: pl.BlockSpec((r, c), lambda n: (0, 0))
    return pl.pallas_call(
        body,
        grid=(grid,),
        in_specs=[
            pl.BlockSpec((BN, x.shape[1]), lambda n: (n, 0)),
            full(*w1.shape),
            pl.BlockSpec((1, HC), lambda n: (0, 0)),
            pl.BlockSpec((1, HC), lambda n: (0, 0)),
            pl.BlockSpec((1, HC), lambda n: (0, 0)),
            full(*w2.shape),
            pl.BlockSpec((1, HC), lambda n: (0, 0)),
            full(*cw.shape),
            pl.BlockSpec((BN, 1), lambda n: (n, 0)),
            pl.BlockSpec((BN, 1), lambda n: (n, 0)),
        ],
        out_specs=[pl.BlockSpec((BN, HC), lambda n: (n, 0))]
        + [pl.BlockSpec((BN, FC), lambda n: (n, 0)) for _ in range(4)],
        out_shape=[jax.ShapeDtypeStruct((NP, HC), jnp.float32)]
        + [jax.ShapeDtypeStruct((NP, FC), jnp.float32) for _ in range(4)],
    )(x, w1, b1, g1, bb1, w2, b2, cw, d0, d1)


def _tc_mid(a0, a1, a2, a3, cb, g, b, idn, w, d0, d1, NP, BN):
    HC = idn.shape[1]
    OC2 = w.shape[1]
    grid = NP // BN
    nyc = OC2 // FC

    def body(a0_ref, a1_ref, a2_ref, a3_ref, cb_ref, g_ref, b_ref, idn_ref,
             w_ref, d0_ref, d1_ref, h_ref, *y_refs):
        agg = jnp.concatenate(
            [a0_ref[...], a1_ref[...], a2_ref[...], a3_ref[...]], axis=-1)
        dinv = _dinv_block(d0_ref, d1_ref)
        h = agg * dinv + cb_ref[...]
        h = jnp.maximum(h, 0.0)
        h = _ln_rows(h, g_ref[...], b_ref[...])
        h = h + idn_ref[...]
        h_ref[...] = h
        y = jnp.dot(h, w_ref[...], preferred_element_type=jnp.float32)
        y = y * dinv
        for kk in range(nyc):
            y_refs[kk][...] = y[:, kk * FC:(kk + 1) * FC]

    return pl.pallas_call(
        body,
        grid=(grid,),
        in_specs=[pl.BlockSpec((BN, FC), lambda n: (n, 0)) for _ in range(4)]
        + [
            pl.BlockSpec((1, HC), lambda n: (0, 0)),
            pl.BlockSpec((1, HC), lambda n: (0, 0)),
            pl.BlockSpec((1, HC), lambda n: (0, 0)),
            pl.BlockSpec((BN, HC), lambda n: (n, 0)),
            pl.BlockSpec(w.shape, lambda n: (0, 0)),
            pl.BlockSpec((BN, 1), lambda n: (n, 0)),
            pl.BlockSpec((BN, 1), lambda n: (n, 0)),
        ],
        out_specs=[pl.BlockSpec((BN, HC), lambda n: (n, 0))]
        + [pl.BlockSpec((BN, FC), lambda n: (n, 0)) for _ in range(nyc)],
        out_shape=[jax.ShapeDtypeStruct((NP, HC), jnp.float32)]
        + [jax.ShapeDtypeStruct((NP, FC), jnp.float32) for _ in range(nyc)],
    )(a0, a1, a2, a3, cb, g, b, idn, w, d0, d1)


def _tc_final(p0, p1, cb, g, b, idn, rw, rb, d0, d1, NP, BN):
    HC = idn.shape[1]
    OC2 = rw.shape[1]
    grid = NP // BN

    def body(p0_ref, p1_ref, cb_ref, g_ref, b_ref, idn_ref, rw_ref, rb_ref,
             d0_ref, d1_ref, o_ref):
        dinv = _dinv_block(d0_ref, d1_ref)
        h = (p0_ref[...] + p1_ref[...]) * dinv + cb_ref[...]
        h = jnp.maximum(h, 0.0)
        h = _ln_rows(h, g_ref[...], b_ref[...])
        res = jnp.dot(idn_ref[...], rw_ref[...],
                      preferred_element_type=jnp.float32)
        o_ref[...] = h + res + rb_ref[...]

    return pl.pallas_call(
        body,
        grid=(grid,),
        in_specs=[
            pl.BlockSpec((BN, OC2), lambda n: (n, 0)),
            pl.BlockSpec((BN, OC2), lambda n: (n, 0)),
            pl.BlockSpec((1, OC2), lambda n: (0, 0)),
            pl.BlockSpec((1, OC2), lambda n: (0, 0)),
            pl.BlockSpec((1, OC2), lambda n: (0, 0)),
            pl.BlockSpec((BN, HC), lambda n: (n, 0)),
            pl.BlockSpec(rw.shape, lambda n: (0, 0)),
            pl.BlockSpec((1, OC2), lambda n: (0, 0)),
            pl.BlockSpec((BN, 1), lambda n: (n, 0)),
            pl.BlockSpec((BN, 1), lambda n: (n, 0)),
        ],
        out_specs=pl.BlockSpec((BN, OC2), lambda n: (n, 0)),
        out_shape=jax.ShapeDtypeStruct((NP, OC2), jnp.float32),
    )(p0, p1, cb, g, b, idn, rw, rb, d0, d1)


def kernel(x, edge_index, p_w1, p_b1, pln_g, pln_b, p_w2, p_b2,
           c1_w, c1_b, ln1_g, ln1_b, c2_w, c2_b, ln2_g, ln2_b,
           c3_w, c3_b, ln3_g, ln3_b, res_w, res_b):
    N, IC = x.shape
    E = edge_index.shape[1]
    BN = 512
    NP = _ceil_to(N, BN)
    E2 = E + NP
    EP = _ceil_to(E2, NC * NS * EC * 4)
    pad = EP - E2

    i32 = jnp.int32
    loop = jnp.arange(NP, dtype=i32)
    spread = N + jnp.arange(pad, dtype=i32) % (NP - N)
    srcs = jnp.concatenate([edge_index[0], loop, spread])
    dsts = jnp.concatenate([edge_index[1], loop, spread])
    x_pad = jnp.pad(x, ((0, NP - N), (0, 0)))

    zrows = jnp.zeros((NP, FC), jnp.float32)
    z1d = jnp.zeros((NP,), jnp.float32)

    b2 = lambda v: v.reshape(1, -1)

    d0f, d1f = _sc_degree(dsts, z1d, NP, EP)
    d0 = d0f.reshape(NP, 1)
    d1 = d1f.reshape(NP, 1)

    idn, *y1 = _tc_proj(x_pad, p_w1, b2(p_b1), b2(pln_g), b2(pln_b),
                        p_w2, b2(p_b2), c1_w, d0, d1, NP, BN)

    a = _sc_agg512(y1, srcs, dsts, zrows, NP, EP)

    idn2, *y2 = _tc_mid(*a, b2(c1_b), b2(ln1_g), b2(ln1_b), idn, c2_w,
                        d0, d1, NP, BN)

    a = _sc_agg512(y2, srcs, dsts, zrows, NP, EP)

    idn3, y3 = _tc_mid(*a, b2(c2_b), b2(ln2_g), b2(ln2_b), idn2, c3_w,
                       d0, d1, NP, BN)

    p0, p1 = _sc_agg128(y3, srcs, dsts, zrows, NP, EP)

    out = _tc_final(p0, p1, b2(c3_b), b2(ln3_g), b2(ln3_b), idn3,
                    res_w, b2(res_b), d0, d1, NP, BN)
    return out[:N]


# speedup vs baseline: 13.9081x; 1.0305x over previous
import functools

import jax
import jax.numpy as jnp
from jax import lax
from jax.experimental import pallas as pl
from jax.experimental.pallas import tpu as pltpu
from jax.experimental.pallas import tpu_sc as plsc

NC = 2
NS = 16
EC = 128
FC = 128


def _mesh():
    return plsc.VectorSubcoreMesh(
        core_axis_name="c", subcore_axis_name="s", num_cores=NC, num_subcores=NS
    )


def _ceil_to(a, m):
    return (a + m - 1) // m * m


def _sc_degree(dsts, z1d, NP, EP):
    EPC = EP // NC
    EPT = EPC // NS
    NCH = EPT // EC
    RPT = NP // NS

    @functools.partial(
        pl.kernel,
        out_type=(
            jax.ShapeDtypeStruct((NP,), jnp.float32),
            jax.ShapeDtypeStruct((NP,), jnp.float32),
        ),
        mesh=_mesh(),
        scratch_types=[
            pltpu.VMEM((EC,), jnp.int32),
            pltpu.VMEM((EC,), jnp.float32),
            pltpu.VMEM_SHARED((NP,), jnp.float32),
        ],
    )
    def k(dst_hbm, z_hbm, out0, out1, didx, ones_v, sdeg):
        cid = lax.axis_index("c")
        sid = lax.axis_index("s")
        for j in range(EC // 16):
            ones_v[pl.ds(j * 16, 16)] = jnp.full((16,), 1.0, jnp.float32)

        @pl.when(sid == 0)
        def _():
            pltpu.sync_copy(z_hbm, sdeg)

        plsc.subcore_barrier()

        def body(i, carry):
            base = cid * EPC + sid * EPT + i * EC
            pltpu.sync_copy(dst_hbm.at[pl.ds(base, EC)], didx)
            pltpu.sync_copy(ones_v, sdeg.at[didx], add=True)
            return carry

        lax.fori_loop(0, NCH, body, 0)
        plsc.subcore_barrier()

        @pl.when(cid == 0)
        def _():
            pltpu.sync_copy(
                sdeg.at[pl.ds(sid * RPT, RPT)], out0.at[pl.ds(sid * RPT, RPT)]
            )

        @pl.when(cid == 1)
        def _():
            pltpu.sync_copy(
                sdeg.at[pl.ds(sid * RPT, RPT)], out1.at[pl.ds(sid * RPT, RPT)]
            )

    return k(dsts, z1d)


def _edge_pipeline(y_hbm, src_hbm, dst_hbm, acc, sidx, didx, rows,
                   isem, gsem, ssem, base0, NCH):
    NB8, TAIL = NCH // 8, NCH % 8

    def idxload(i, s):
        base = base0 + i * EC
        pltpu.async_copy(src_hbm.at[pl.ds(base, EC)], sidx.at[s], isem[s])
        pltpu.async_copy(dst_hbm.at[pl.ds(base, EC)], didx.at[s], isem[s])

    def idxwait(s):
        pltpu.make_async_copy(
            src_hbm.at[pl.ds(0, EC)], sidx.at[s], isem[s]).wait()
        pltpu.make_async_copy(
            dst_hbm.at[pl.ds(0, EC)], didx.at[s], isem[s]).wait()

    def gather(s, p):
        pltpu.async_copy(y_hbm.at[sidx.at[s]], rows.at[p], gsem[p])

    def gatherwait(p):
        pltpu.make_async_copy(
            y_hbm.at[sidx.at[0]], rows.at[p], gsem[p]).wait()

    def scatter(s, p):
        pltpu.async_copy(rows.at[p], acc.at[didx.at[s]], ssem[p], add=True)

    def scatterwait(p):
        pltpu.make_async_copy(
            rows.at[p], acc.at[didx.at[0]], ssem[p]).wait()

    for s in range(6):
        idxload(jnp.int32(s), s)
    idxwait(0)
    gather(0, 0)

    def substep(i, b, static):
        s6, s1 = (b + 6) % 8, (b + 1) % 8
        p, p1 = b % 2, (b + 1) % 2

        def guard(cond, fn):
            if static:
                if cond:
                    fn()
            else:
                pl.when(cond)(fn)

        guard(i + 6 < NCH, lambda: idxload(i + 6, s6))

        def prep_next():
            idxwait(s1)
            guard(i >= 1, lambda: scatterwait(p1))
            gather(s1, p1)

        guard(i + 1 < NCH, prep_next)
        gatherwait(p)
        scatter(b, p)

    def body(j, carry):
        for b in range(8):
            substep(j * 8 + b, b, False)
        return carry

    lax.fori_loop(0, NB8, body, 0)
    for b in range(TAIL):
        substep(NB8 * 8 + b, b, True)
    scatterwait(0)
    scatterwait(1)


def _sc_agg512(y_chunks, srcs, dsts, zrows, NP, EP):
    EPT = EP // NS
    NCH = EPT // EC
    RPT = NP // NS

    @functools.partial(
        pl.kernel,
        out_type=tuple(
            jax.ShapeDtypeStruct((NP, FC), jnp.float32) for _ in range(4)
        ),
        mesh=_mesh(),
        scratch_types=[
            pltpu.VMEM((8, EC), jnp.int32),
            pltpu.VMEM((8, EC), jnp.int32),
            pltpu.VMEM((2, EC, FC), jnp.float32),
            pltpu.VMEM_SHARED((NP, FC), jnp.float32),
            [pltpu.SemaphoreType.DMA] * 8,
            [pltpu.SemaphoreType.DMA] * 2,
            [pltpu.SemaphoreType.DMA] * 2,
        ],
    )
    def k(y0, y1, y2, y3, src_hbm, dst_hbm, z_hbm, o0, o1, o2, o3,
          sidx, didx, rows, acc, isem, gsem, ssem):
        cid = lax.axis_index("c")
        sid = lax.axis_index("s")

        def one_chunk(y_hbm, out_hbm):
            pltpu.sync_copy(
                z_hbm.at[pl.ds(sid * RPT, RPT)], acc.at[pl.ds(sid * RPT, RPT)]
            )
            plsc.subcore_barrier()
            _edge_pipeline(y_hbm, src_hbm, dst_hbm, acc, sidx, didx, rows,
                           isem, gsem, ssem, sid * EPT, NCH)
            plsc.subcore_barrier()
            pltpu.sync_copy(
                acc.at[pl.ds(sid * RPT, RPT)],
                out_hbm.at[pl.ds(sid * RPT, RPT)],
            )
            plsc.subcore_barrier()

        @pl.when(cid == 0)
        def _():
            one_chunk(y0, o0)
            one_chunk(y1, o1)

        @pl.when(cid == 1)
        def _():
            one_chunk(y2, o2)
            one_chunk(y3, o3)

    return k(*y_chunks, srcs, dsts, zrows)


def _sc_agg128(y, srcs, dsts, zrows, NP, EP):
    EPC = EP // NC
    EPT = EPC // NS
    NCH = EPT // EC
    RPT = NP // NS

    @functools.partial(
        pl.kernel,
        out_type=(
            jax.ShapeDtypeStruct((NP, FC), jnp.float32),
            jax.ShapeDtypeStruct((NP, FC), jnp.float32),
        ),
        mesh=_mesh(),
        scratch_types=[
            pltpu.VMEM((8, EC), jnp.int32),
            pltpu.VMEM((8, EC), jnp.int32),
            pltpu.VMEM((2, EC, FC), jnp.float32),
            pltpu.VMEM_SHARED((NP, FC), jnp.float32),
            [pltpu.SemaphoreType.DMA] * 8,
            [pltpu.SemaphoreType.DMA] * 2,
            [pltpu.SemaphoreType.DMA] * 2,
        ],
    )
    def k(y_hbm, src_hbm, dst_hbm, z_hbm, o0, o1,
          sidx, didx, rows, acc, isem, gsem, ssem):
        cid = lax.axis_index("c")
        sid = lax.axis_index("s")
        pltpu.sync_copy(
            z_hbm.at[pl.ds(sid * RPT, RPT)], acc.at[pl.ds(sid * RPT, RPT)]
        )
        plsc.subcore_barrier()
        _edge_pipeline(y_hbm, src_hbm, dst_hbm, acc, sidx, didx, rows,
                       isem, gsem, ssem, cid * EPC + sid * EPT, NCH)
        plsc.subcore_barrier()

        @pl.when(cid == 0)
        def _():
            pltpu.sync_copy(
                acc.at[pl.ds(sid * RPT, RPT)], o0.at[pl.ds(sid * RPT, RPT)]
            )

        @pl.when(cid == 1)
        def _():
            pltpu.sync_copy(
                acc.at[pl.ds(sid * RPT, RPT)], o1.at[pl.ds(sid * RPT, RPT)]
            )

    return k(y, srcs, dsts, zrows)


def _ln_rows(h, g, b, eps=1e-5):
    m = jnp.mean(h, axis=-1, keepdims=True)
    v = jnp.mean((h - m) * (h - m), axis=-1, keepdims=True)
    return (h - m) * lax.rsqrt(v + eps) * g + b


def _dinv_block(d0, d1):
    return lax.rsqrt(d0[...] + d1[...])


def _tc_proj(x, w1, b1, g1, bb1, w2, b2, NP, BN):
    HC = w1.shape[1]
    grid = NP // BN

    def body(x_ref, w1_ref, b1_ref, g1_ref, bb1_ref, w2_ref, b2_ref, idn_ref):
        h = jnp.dot(x_ref[...], w1_ref[...], preferred_element_type=jnp.float32)
        h = jnp.maximum(h + b1_ref[...], 0.0)
        h = _ln_rows(h, g1_ref[...], bb1_ref[...])
        h = jnp.dot(h, w2_ref[...], preferred_element_type=jnp.float32)
        idn_ref[...] = h + b2_ref[...]

    full = lambda r, c: pl.BlockSpec((r, c), lambda n: (0, 0))
    return pl.pallas_call(
        body,
        grid=(grid,),
        in_specs=[
            pl.BlockSpec((BN, x.shape[1]), lambda n: (n, 0)),
            full(*w1.shape),
            pl.BlockSpec((1, HC), lambda n: (0, 0)),
            pl.BlockSpec((1, HC), lambda n: (0, 0)),
            pl.BlockSpec((1, HC), lambda n: (0, 0)),
            full(*w2.shape),
            pl.BlockSpec((1, HC), lambda n: (0, 0)),
        ],
        out_specs=pl.BlockSpec((BN, HC), lambda n: (n, 0)),
        out_shape=jax.ShapeDtypeStruct((NP, HC), jnp.float32),
    )(x, w1, b1, g1, bb1, w2, b2)


def _tc_ymul(idn, w, d0, d1, NP, BN):
    HC = idn.shape[1]
    nyc = w.shape[1] // FC
    grid = NP // BN

    def body(idn_ref, w_ref, d0_ref, d1_ref, *y_refs):
        dinv = _dinv_block(d0_ref, d1_ref)
        y = jnp.dot(idn_ref[...], w_ref[...],
                    preferred_element_type=jnp.float32)
        y = y * dinv
        for kk in range(nyc):
            y_refs[kk][...] = y[:, kk * FC:(kk + 1) * FC]

    return pl.pallas_call(
        body,
        grid=(grid,),
        in_specs=[
            pl.BlockSpec((BN, HC), lambda n: (n, 0)),
            pl.BlockSpec(w.shape, lambda n: (0, 0)),
            pl.BlockSpec((BN, 1), lambda n: (n, 0)),
            pl.BlockSpec((BN, 1), lambda n: (n, 0)),
        ],
        out_specs=[pl.BlockSpec((BN, FC), lambda n: (n, 0))
                   for _ in range(nyc)],
        out_shape=[jax.ShapeDtypeStruct((NP, FC), jnp.float32)
                   for _ in range(nyc)],
    )(idn, w, d0, d1)


def _tc_mid(a0, a1, a2, a3, cb, g, b, idn, w, d0, d1, NP, BN):
    HC = idn.shape[1]
    OC2 = w.shape[1]
    grid = NP // BN
    nyc = OC2 // FC

    def body(a0_ref, a1_ref, a2_ref, a3_ref, cb_ref, g_ref, b_ref, idn_ref,
             w_ref, d0_ref, d1_ref, h_ref, *y_refs):
        agg = jnp.concatenate(
            [a0_ref[...], a1_ref[...], a2_ref[...], a3_ref[...]], axis=-1)
        dinv = _dinv_block(d0_ref, d1_ref)
        h = agg * dinv + cb_ref[...]
        h = jnp.maximum(h, 0.0)
        h = _ln_rows(h, g_ref[...], b_ref[...])
        h = h + idn_ref[...]
        h_ref[...] = h
        y = jnp.dot(h, w_ref[...], preferred_element_type=jnp.float32)
        y = y * dinv
        for kk in range(nyc):
            y_refs[kk][...] = y[:, kk * FC:(kk + 1) * FC]

    return pl.pallas_call(
        body,
        grid=(grid,),
        in_specs=[pl.BlockSpec((BN, FC), lambda n: (n, 0)) for _ in range(4)]
        + [
            pl.BlockSpec((1, HC), lambda n: (0, 0)),
            pl.BlockSpec((1, HC), lambda n: (0, 0)),
            pl.BlockSpec((1, HC), lambda n: (0, 0)),
            pl.BlockSpec((BN, HC), lambda n: (n, 0)),
            pl.BlockSpec(w.shape, lambda n: (0, 0)),
            pl.BlockSpec((BN, 1), lambda n: (n, 0)),
            pl.BlockSpec((BN, 1), lambda n: (n, 0)),
        ],
        out_specs=[pl.BlockSpec((BN, HC), lambda n: (n, 0))]
        + [pl.BlockSpec((BN, FC), lambda n: (n, 0)) for _ in range(nyc)],
        out_shape=[jax.ShapeDtypeStruct((NP, HC), jnp.float32)]
        + [jax.ShapeDtypeStruct((NP, FC), jnp.float32) for _ in range(nyc)],
    )(a0, a1, a2, a3, cb, g, b, idn, w, d0, d1)


def _tc_res(idn, rw, rb, NP, BN):
    HC = idn.shape[1]
    OC2 = rw.shape[1]
    grid = NP // BN

    def body(idn_ref, rw_ref, rb_ref, o_ref):
        o_ref[...] = jnp.dot(idn_ref[...], rw_ref[...],
                             preferred_element_type=jnp.float32) + rb_ref[...]

    return pl.pallas_call(
        body,
        grid=(grid,),
        in_specs=[
            pl.BlockSpec((BN, HC), lambda n: (n, 0)),
            pl.BlockSpec(rw.shape, lambda n: (0, 0)),
            pl.BlockSpec((1, OC2), lambda n: (0, 0)),
        ],
        out_specs=pl.BlockSpec((BN, OC2), lambda n: (n, 0)),
        out_shape=jax.ShapeDtypeStruct((NP, OC2), jnp.float32),
    )(idn, rw, rb)


def _tc_final(p0, p1, cb, g, b, res, d0, d1, NP, BN):
    OC2 = res.shape[1]
    grid = NP // BN

    def body(p0_ref, p1_ref, cb_ref, g_ref, b_ref, res_ref,
             d0_ref, d1_ref, o_ref):
        dinv = _dinv_block(d0_ref, d1_ref)
        h = (p0_ref[...] + p1_ref[...]) * dinv + cb_ref[...]
        h = jnp.maximum(h, 0.0)
        h = _ln_rows(h, g_ref[...], b_ref[...])
        o_ref[...] = h + res_ref[...]

    return pl.pallas_call(
        body,
        grid=(grid,),
        in_specs=[
            pl.BlockSpec((BN, OC2), lambda n: (n, 0)),
            pl.BlockSpec((BN, OC2), lambda n: (n, 0)),
            pl.BlockSpec((1, OC2), lambda n: (0, 0)),
            pl.BlockSpec((1, OC2), lambda n: (0, 0)),
            pl.BlockSpec((1, OC2), lambda n: (0, 0)),
            pl.BlockSpec((BN, OC2), lambda n: (n, 0)),
            pl.BlockSpec((BN, 1), lambda n: (n, 0)),
            pl.BlockSpec((BN, 1), lambda n: (n, 0)),
        ],
        out_specs=pl.BlockSpec((BN, OC2), lambda n: (n, 0)),
        out_shape=jax.ShapeDtypeStruct((NP, OC2), jnp.float32),
    )(p0, p1, cb, g, b, res, d0, d1)


def kernel(x, edge_index, p_w1, p_b1, pln_g, pln_b, p_w2, p_b2,
           c1_w, c1_b, ln1_g, ln1_b, c2_w, c2_b, ln2_g, ln2_b,
           c3_w, c3_b, ln3_g, ln3_b, res_w, res_b):
    N, IC = x.shape
    E = edge_index.shape[1]
    BN = 512
    NP = _ceil_to(N, BN)
    E2 = E + NP
    EP = _ceil_to(E2, NC * NS * EC)
    pad = EP - E2

    i32 = jnp.int32
    loop = jnp.arange(NP, dtype=i32)
    spread = N + jnp.arange(pad, dtype=i32) % (NP - N)
    srcs = jnp.concatenate([edge_index[0], loop, spread])
    dsts = jnp.concatenate([edge_index[1], loop, spread])
    x_pad = jnp.pad(x, ((0, NP - N), (0, 0)))

    zrows = jnp.zeros((NP, FC), jnp.float32)
    z1d = jnp.zeros((NP,), jnp.float32)

    b2 = lambda v: v.reshape(1, -1)

    d0f, d1f = _sc_degree(dsts, z1d, NP, EP)
    d0 = d0f.reshape(NP, 1)
    d1 = d1f.reshape(NP, 1)

    idn = _tc_proj(x_pad, p_w1, b2(p_b1), b2(pln_g), b2(pln_b),
                   p_w2, b2(p_b2), NP, BN)
    y1 = _tc_ymul(idn, c1_w, d0, d1, NP, BN)

    a = _sc_agg512(y1, srcs, dsts, zrows, NP, EP)

    idn2, *y2 = _tc_mid(*a, b2(c1_b), b2(ln1_g), b2(ln1_b), idn, c2_w,
                        d0, d1, NP, BN)

    a = _sc_agg512(y2, srcs, dsts, zrows, NP, EP)

    idn3, y3 = _tc_mid(*a, b2(c2_b), b2(ln2_g), b2(ln2_b), idn2, c3_w,
                       d0, d1, NP, BN)

    p0, p1 = _sc_agg128(y3, srcs, dsts, zrows, NP, EP)
    res = _tc_res(idn3, res_w, b2(res_b), NP, BN)

    out = _tc_final(p0, p1, b2(c3_b), b2(ln3_g), b2(ln3_b), res,
                    d0, d1, NP, BN)
    return out[:N]


# speedup vs baseline: 14.2514x; 1.0247x over previous
import functools

import jax
import jax.numpy as jnp
from jax import lax
from jax.experimental import pallas as pl
from jax.experimental.pallas import tpu as pltpu
from jax.experimental.pallas import tpu_sc as plsc

NC = 2
NS = 16
EC = 128
FC = 128


def _mesh():
    return plsc.VectorSubcoreMesh(
        core_axis_name="c", subcore_axis_name="s", num_cores=NC, num_subcores=NS
    )


def _ceil_to(a, m):
    return (a + m - 1) // m * m


def _sc_degree(dsts, z1d, NP, EP):
    EPC = EP // NC
    EPT = EPC // NS
    NCH = EPT // EC
    RPT = NP // NS
    NB4, TAIL = NCH // 4, NCH % 4

    @functools.partial(
        pl.kernel,
        out_type=(
            jax.ShapeDtypeStruct((NP,), jnp.float32),
            jax.ShapeDtypeStruct((NP,), jnp.float32),
        ),
        mesh=_mesh(),
        scratch_types=[
            pltpu.VMEM((4, EC), jnp.int32),
            pltpu.VMEM((EC,), jnp.float32),
            pltpu.VMEM_SHARED((NP,), jnp.float32),
            [pltpu.SemaphoreType.DMA] * 4,
            [pltpu.SemaphoreType.DMA] * 2,
        ],
    )
    def k(dst_hbm, z_hbm, out0, out1, didx, ones_v, sdeg, isem, ssem):
        cid = lax.axis_index("c")
        sid = lax.axis_index("s")
        base0 = cid * EPC + sid * EPT
        for j in range(EC // 16):
            ones_v[pl.ds(j * 16, 16)] = jnp.full((16,), 1.0, jnp.float32)

        def idxload(i, s):
            pltpu.async_copy(
                dst_hbm.at[pl.ds(base0 + i * EC, EC)], didx.at[s], isem[s])

        def idxwait(s):
            pltpu.make_async_copy(
                dst_hbm.at[pl.ds(0, EC)], didx.at[s], isem[s]).wait()

        def scatter(s, p):
            pltpu.async_copy(ones_v, sdeg.at[didx.at[s]], ssem[p], add=True)

        def scatterwait(p):
            pltpu.make_async_copy(
                ones_v, sdeg.at[didx.at[0]], ssem[p]).wait()

        for s in range(2):
            idxload(jnp.int32(s), s)

        @pl.when(sid == 0)
        def _():
            pltpu.sync_copy(z_hbm, sdeg)

        plsc.subcore_barrier()

        def substep(i, b, static):
            p = b % 2

            def guard(cond, fn):
                if static:
                    if cond:
                        fn()
                else:
                    pl.when(cond)(fn)

            idxwait(b)
            guard(i >= 2, lambda: scatterwait(p))
            scatter(b, p)
            guard(i + 2 < NCH, lambda: idxload(i + 2, (b + 2) % 4))

        def body(j, carry):
            for b in range(4):
                substep(j * 4 + b, b, False)
            return carry

        lax.fori_loop(0, NB4, body, 0)
        for b in range(TAIL):
            substep(NB4 * 4 + b, b, True)
        scatterwait(0)
        scatterwait(1)
        plsc.subcore_barrier()

        @pl.when(cid == 0)
        def _():
            pltpu.sync_copy(
                sdeg.at[pl.ds(sid * RPT, RPT)], out0.at[pl.ds(sid * RPT, RPT)]
            )

        @pl.when(cid == 1)
        def _():
            pltpu.sync_copy(
                sdeg.at[pl.ds(sid * RPT, RPT)], out1.at[pl.ds(sid * RPT, RPT)]
            )

    return k(dsts, z1d)


def _edge_pipeline(y_hbm, src_hbm, dst_hbm, acc, sidx, didx, rows,
                   isem, gsem, ssem, base0, NCH):
    NB8, TAIL = NCH // 8, NCH % 8

    def idxload(i, s):
        base = base0 + i * EC
        pltpu.async_copy(src_hbm.at[pl.ds(base, EC)], sidx.at[s], isem[s])
        pltpu.async_copy(dst_hbm.at[pl.ds(base, EC)], didx.at[s], isem[s])

    def idxwait(s):
        pltpu.make_async_copy(
            src_hbm.at[pl.ds(0, EC)], sidx.at[s], isem[s]).wait()
        pltpu.make_async_copy(
            dst_hbm.at[pl.ds(0, EC)], didx.at[s], isem[s]).wait()

    def gather(s, p):
        pltpu.async_copy(y_hbm.at[sidx.at[s]], rows.at[p], gsem[p])

    def gatherwait(p):
        pltpu.make_async_copy(
            y_hbm.at[sidx.at[0]], rows.at[p], gsem[p]).wait()

    def scatter(s, p):
        pltpu.async_copy(rows.at[p], acc.at[didx.at[s]], ssem[p], add=True)

    def scatterwait(p):
        pltpu.make_async_copy(
            rows.at[p], acc.at[didx.at[0]], ssem[p]).wait()

    for s in range(6):
        idxload(jnp.int32(s), s)
    idxwait(0)
    gather(0, 0)

    def substep(i, b, static):
        s6, s1 = (b + 6) % 8, (b + 1) % 8
        p, p1 = b % 2, (b + 1) % 2

        def guard(cond, fn):
            if static:
                if cond:
                    fn()
            else:
                pl.when(cond)(fn)

        guard(i + 6 < NCH, lambda: idxload(i + 6, s6))

        def prep_next():
            idxwait(s1)
            guard(i >= 1, lambda: scatterwait(p1))
            gather(s1, p1)

        guard(i + 1 < NCH, prep_next)
        gatherwait(p)
        scatter(b, p)

    def body(j, carry):
        for b in range(8):
            substep(j * 8 + b, b, False)
        return carry

    lax.fori_loop(0, NB8, body, 0)
    for b in range(TAIL):
        substep(NB8 * 8 + b, b, True)
    scatterwait(0)
    scatterwait(1)


def _sc_agg512(y_chunks, srcs, dsts, zrows, NP, EP):
    EPT = EP // NS
    NCH = EPT // EC
    RPT = NP // NS

    @functools.partial(
        pl.kernel,
        out_type=tuple(
            jax.ShapeDtypeStruct((NP, FC), jnp.float32) for _ in range(4)
        ),
        mesh=_mesh(),
        scratch_types=[
            pltpu.VMEM((8, EC), jnp.int32),
            pltpu.VMEM((8, EC), jnp.int32),
            pltpu.VMEM((2, EC, FC), jnp.float32),
            pltpu.VMEM_SHARED((NP, FC), jnp.float32),
            [pltpu.SemaphoreType.DMA] * 8,
            [pltpu.SemaphoreType.DMA] * 2,
            [pltpu.SemaphoreType.DMA] * 2,
        ],
    )
    def k(y0, y1, y2, y3, src_hbm, dst_hbm, z_hbm, o0, o1, o2, o3,
          sidx, didx, rows, acc, isem, gsem, ssem):
        cid = lax.axis_index("c")
        sid = lax.axis_index("s")

        stripe = pl.ds(sid * RPT, RPT)

        def one_chunk(y_hbm, out_hbm, last):
            _edge_pipeline(y_hbm, src_hbm, dst_hbm, acc, sidx, didx, rows,
                           isem, gsem, ssem, sid * EPT, NCH)
            plsc.subcore_barrier()
            pltpu.async_copy(acc.at[stripe], out_hbm.at[stripe], gsem[0])
            if not last:
                pltpu.async_copy(z_hbm.at[stripe], acc.at[stripe], gsem[1])
            pltpu.make_async_copy(
                acc.at[stripe], out_hbm.at[stripe], gsem[0]).wait()
            if not last:
                pltpu.make_async_copy(
                    z_hbm.at[stripe], acc.at[stripe], gsem[1]).wait()
                plsc.subcore_barrier()

        pltpu.sync_copy(z_hbm.at[stripe], acc.at[stripe])
        plsc.subcore_barrier()

        @pl.when(cid == 0)
        def _():
            one_chunk(y0, o0, False)
            one_chunk(y1, o1, True)

        @pl.when(cid == 1)
        def _():
            one_chunk(y2, o2, False)
            one_chunk(y3, o3, True)

    return k(*y_chunks, srcs, dsts, zrows)


def _sc_agg128(y, srcs, dsts, zrows, NP, EP):
    EPC = EP // NC
    EPT = EPC // NS
    NCH = EPT // EC
    RPT = NP // NS

    @functools.partial(
        pl.kernel,
        out_type=(
            jax.ShapeDtypeStruct((NP, FC), jnp.float32),
            jax.ShapeDtypeStruct((NP, FC), jnp.float32),
        ),
        mesh=_mesh(),
        scratch_types=[
            pltpu.VMEM((8, EC), jnp.int32),
            pltpu.VMEM((8, EC), jnp.int32),
            pltpu.VMEM((2, EC, FC), jnp.float32),
            pltpu.VMEM_SHARED((NP, FC), jnp.float32),
            [pltpu.SemaphoreType.DMA] * 8,
            [pltpu.SemaphoreType.DMA] * 2,
            [pltpu.SemaphoreType.DMA] * 2,
        ],
    )
    def k(y_hbm, src_hbm, dst_hbm, z_hbm, o0, o1,
          sidx, didx, rows, acc, isem, gsem, ssem):
        cid = lax.axis_index("c")
        sid = lax.axis_index("s")
        pltpu.sync_copy(
            z_hbm.at[pl.ds(sid * RPT, RPT)], acc.at[pl.ds(sid * RPT, RPT)]
        )
        plsc.subcore_barrier()
        _edge_pipeline(y_hbm, src_hbm, dst_hbm, acc, sidx, didx, rows,
                       isem, gsem, ssem, cid * EPC + sid * EPT, NCH)
        plsc.subcore_barrier()

        @pl.when(cid == 0)
        def _():
            pltpu.sync_copy(
                acc.at[pl.ds(sid * RPT, RPT)], o0.at[pl.ds(sid * RPT, RPT)]
            )

        @pl.when(cid == 1)
        def _():
            pltpu.sync_copy(
                acc.at[pl.ds(sid * RPT, RPT)], o1.at[pl.ds(sid * RPT, RPT)]
            )

    return k(y, srcs, dsts, zrows)


def _ln_rows(h, g, b, eps=1e-5):
    m = jnp.mean(h, axis=-1, keepdims=True)
    v = jnp.mean((h - m) * (h - m), axis=-1, keepdims=True)
    return (h - m) * lax.rsqrt(v + eps) * g + b


def _dinv_block(d0, d1):
    return lax.rsqrt(d0[...] + d1[...])


def _tc_proj(x, w1, b1, g1, bb1, w2, b2, NP, BN):
    HC = w1.shape[1]
    grid = NP // BN

    def body(x_ref, w1_ref, b1_ref, g1_ref, bb1_ref, w2_ref, b2_ref, idn_ref):
        h = jnp.dot(x_ref[...], w1_ref[...], preferred_element_type=jnp.float32)
        h = jnp.maximum(h + b1_ref[...], 0.0)
        h = _ln_rows(h, g1_ref[...], bb1_ref[...])
        h = jnp.dot(h, w2_ref[...], preferred_element_type=jnp.float32)
        idn_ref[...] = h + b2_ref[...]

    full = lambda r, c: pl.BlockSpec((r, c), lambda n: (0, 0))
    return pl.pallas_call(
        body,
        grid=(grid,),
        in_specs=[
            pl.BlockSpec((BN, x.shape[1]), lambda n: (n, 0)),
            full(*w1.shape),
            pl.BlockSpec((1, HC), lambda n: (0, 0)),
            pl.BlockSpec((1, HC), lambda n: (0, 0)),
            pl.BlockSpec((1, HC), lambda n: (0, 0)),
            full(*w2.shape),
            pl.BlockSpec((1, HC), lambda n: (0, 0)),
        ],
        out_specs=pl.BlockSpec((BN, HC), lambda n: (n, 0)),
        out_shape=jax.ShapeDtypeStruct((NP, HC), jnp.float32),
    )(x, w1, b1, g1, bb1, w2, b2)


def _tc_ymul(idn, w, d0, d1, NP, BN):
    HC = idn.shape[1]
    nyc = w.shape[1] // FC
    grid = NP // BN

    def body(idn_ref, w_ref, d0_ref, d1_ref, *y_refs):
        dinv = _dinv_block(d0_ref, d1_ref)
        y = jnp.dot(idn_ref[...], w_ref[...],
                    preferred_element_type=jnp.float32)
        y = y * dinv
        for kk in range(nyc):
            y_refs[kk][...] = y[:, kk * FC:(kk + 1) * FC]

    return pl.pallas_call(
        body,
        grid=(grid,),
        in_specs=[
            pl.BlockSpec((BN, HC), lambda n: (n, 0)),
            pl.BlockSpec(w.shape, lambda n: (0, 0)),
            pl.BlockSpec((BN, 1), lambda n: (n, 0)),
            pl.BlockSpec((BN, 1), lambda n: (n, 0)),
        ],
        out_specs=[pl.BlockSpec((BN, FC), lambda n: (n, 0))
                   for _ in range(nyc)],
        out_shape=[jax.ShapeDtypeStruct((NP, FC), jnp.float32)
                   for _ in range(nyc)],
    )(idn, w, d0, d1)


def _tc_mid(a0, a1, a2, a3, cb, g, b, idn, w, d0, d1, NP, BN):
    HC = idn.shape[1]
    OC2 = w.shape[1]
    grid = NP // BN
    nyc = OC2 // FC

    def body(a0_ref, a1_ref, a2_ref, a3_ref, cb_ref, g_ref, b_ref, idn_ref,
             w_ref, d0_ref, d1_ref, h_ref, *y_refs):
        agg = jnp.concatenate(
            [a0_ref[...], a1_ref[...], a2_ref[...], a3_ref[...]], axis=-1)
        dinv = _dinv_block(d0_ref, d1_ref)
        h = agg * dinv + cb_ref[...]
        h = jnp.maximum(h, 0.0)
        h = _ln_rows(h, g_ref[...], b_ref[...])
        h = h + idn_ref[...]
        h_ref[...] = h
        y = jnp.dot(h, w_ref[...], preferred_element_type=jnp.float32)
        y = y * dinv
        for kk in range(nyc):
            y_refs[kk][...] = y[:, kk * FC:(kk + 1) * FC]

    return pl.pallas_call(
        body,
        grid=(grid,),
        in_specs=[pl.BlockSpec((BN, FC), lambda n: (n, 0)) for _ in range(4)]
        + [
            pl.BlockSpec((1, HC), lambda n: (0, 0)),
            pl.BlockSpec((1, HC), lambda n: (0, 0)),
            pl.BlockSpec((1, HC), lambda n: (0, 0)),
            pl.BlockSpec((BN, HC), lambda n: (n, 0)),
            pl.BlockSpec(w.shape, lambda n: (0, 0)),
            pl.BlockSpec((BN, 1), lambda n: (n, 0)),
            pl.BlockSpec((BN, 1), lambda n: (n, 0)),
        ],
        out_specs=[pl.BlockSpec((BN, HC), lambda n: (n, 0))]
        + [pl.BlockSpec((BN, FC), lambda n: (n, 0)) for _ in range(nyc)],
        out_shape=[jax.ShapeDtypeStruct((NP, HC), jnp.float32)]
        + [jax.ShapeDtypeStruct((NP, FC), jnp.float32) for _ in range(nyc)],
    )(a0, a1, a2, a3, cb, g, b, idn, w, d0, d1)


def _tc_res(idn, rw, rb, NP, BN):
    HC = idn.shape[1]
    OC2 = rw.shape[1]
    grid = NP // BN

    def body(idn_ref, rw_ref, rb_ref, o_ref):
        o_ref[...] = jnp.dot(idn_ref[...], rw_ref[...],
                             preferred_element_type=jnp.float32) + rb_ref[...]

    return pl.pallas_call(
        body,
        grid=(grid,),
        in_specs=[
            pl.BlockSpec((BN, HC), lambda n: (n, 0)),
            pl.BlockSpec(rw.shape, lambda n: (0, 0)),
            pl.BlockSpec((1, OC2), lambda n: (0, 0)),
        ],
        out_specs=pl.BlockSpec((BN, OC2), lambda n: (n, 0)),
        out_shape=jax.ShapeDtypeStruct((NP, OC2), jnp.float32),
    )(idn, rw, rb)


def _tc_final(p0, p1, cb, g, b, res, d0, d1, N, NP, BN):
    OC2 = res.shape[1]
    grid = NP // BN

    def body(p0_ref, p1_ref, cb_ref, g_ref, b_ref, res_ref,
             d0_ref, d1_ref, o_ref):
        dinv = _dinv_block(d0_ref, d1_ref)
        h = (p0_ref[...] + p1_ref[...]) * dinv + cb_ref[...]
        h = jnp.maximum(h, 0.0)
        h = _ln_rows(h, g_ref[...], b_ref[...])
        o_ref[...] = h + res_ref[...]

    return pl.pallas_call(
        body,
        grid=(grid,),
        in_specs=[
            pl.BlockSpec((BN, OC2), lambda n: (n, 0)),
            pl.BlockSpec((BN, OC2), lambda n: (n, 0)),
            pl.BlockSpec((1, OC2), lambda n: (0, 0)),
            pl.BlockSpec((1, OC2), lambda n: (0, 0)),
            pl.BlockSpec((1, OC2), lambda n: (0, 0)),
            pl.BlockSpec((BN, OC2), lambda n: (n, 0)),
            pl.BlockSpec((BN, 1), lambda n: (n, 0)),
            pl.BlockSpec((BN, 1), lambda n: (n, 0)),
        ],
        out_specs=pl.BlockSpec((BN, OC2), lambda n: (n, 0)),
        out_shape=jax.ShapeDtypeStruct((N, OC2), jnp.float32),
    )(p0, p1, cb, g, b, res, d0, d1)


def kernel(x, edge_index, p_w1, p_b1, pln_g, pln_b, p_w2, p_b2,
           c1_w, c1_b, ln1_g, ln1_b, c2_w, c2_b, ln2_g, ln2_b,
           c3_w, c3_b, ln3_g, ln3_b, res_w, res_b):
    N, IC = x.shape
    E = edge_index.shape[1]
    BN = 512
    NP = _ceil_to(N, BN)
    E2 = E + NP
    EP = _ceil_to(E2, NC * NS * EC)
    pad = EP - E2

    i32 = jnp.int32
    loop = jnp.arange(NP, dtype=i32)
    spread = N + jnp.arange(pad, dtype=i32) % (NP - N)
    srcs = jnp.concatenate([edge_index[0], loop, spread])
    dsts = jnp.concatenate([edge_index[1], loop, spread])
    x_pad = jnp.pad(x, ((0, NP - N), (0, 0)))

    zrows = jnp.zeros((NP, FC), jnp.float32)
    z1d = jnp.zeros((NP,), jnp.float32)

    b2 = lambda v: v.reshape(1, -1)

    d0f, d1f = _sc_degree(dsts, z1d, NP, EP)
    d0 = d0f.reshape(NP, 1)
    d1 = d1f.reshape(NP, 1)

    idn = _tc_proj(x_pad, p_w1, b2(p_b1), b2(pln_g), b2(pln_b),
                   p_w2, b2(p_b2), NP, BN)
    y1 = _tc_ymul(idn, c1_w, d0, d1, NP, BN)

    a = _sc_agg512(y1, srcs, dsts, zrows, NP, EP)

    idn2, *y2 = _tc_mid(*a, b2(c1_b), b2(ln1_g), b2(ln1_b), idn, c2_w,
                        d0, d1, NP, BN)

    a = _sc_agg512(y2, srcs, dsts, zrows, NP, EP)

    idn3, y3 = _tc_mid(*a, b2(c2_b), b2(ln2_g), b2(ln2_b), idn2, c3_w,
                       d0, d1, NP, BN)

    p0, p1 = _sc_agg128(y3, srcs, dsts, zrows, NP, EP)
    res = _tc_res(idn3, res_w, b2(res_b), NP, BN)

    out = _tc_final(p0, p1, b2(c3_b), b2(ln3_g), b2(ln3_b), res,
                    d0, d1, N, NP, BN)
    return out
